# Initial kernel scaffold; baseline (speedup 1.0000x reference)
import jax
import jax.numpy as jnp
from jax.experimental import pallas as pl


def kernel(in_feat, edge_index, W1, al1, ar1, b1, W2, al2, ar2, b2, W3, al3, ar3, b3):
    raise NotImplementedError("write your pallas kernel here")


# speedup vs baseline: 8.0815x; 8.0815x over previous
import functools

import jax
import jax.numpy as jnp
from jax import lax
from jax.experimental import pallas as pl
from jax.experimental.pallas import tpu as pltpu
from jax.experimental.pallas import tpu_sc as plsc

N = 10000
E = 160000
D = 256
DH = D // 2
NP = 10240
NC = 2
NS = 16
HALF = NP // NC
ROWS_T = HALF // NS
EC = E // NS
SUB = 2000
NSUB = EC // SUB
K = 80
NK = SUB // K
MM_BLK = 256
G = 40


def _mm_body(x_ref, w_ref, al_ref, ar_ref, hlo_ref, hhi_ref, el_ref, er_ref):
    x = x_ref[...]
    h = jnp.dot(x, w_ref[...], preferred_element_type=jnp.float32)
    hlo_ref[...] = h[:, :DH]
    hhi_ref[...] = h[:, DH:]
    al = al_ref[...].reshape(1, D)
    ar = ar_ref[...].reshape(1, D)
    el_ref[...] = jnp.sum(h * al, axis=1).reshape(1, 1, D)
    er_ref[...] = jnp.sum(h * ar, axis=1).reshape(1, 1, D)


def _tc_project(x, W, al, ar):
    hlo, hhi, el3, er3 = pl.pallas_call(
        _mm_body,
        grid=(G,),
        in_specs=[
            pl.BlockSpec((MM_BLK, D), lambda i: (i, 0)),
            pl.BlockSpec((D, D), lambda i: (0, 0)),
            pl.BlockSpec((1, 1, D), lambda i: (0, 0, 0)),
            pl.BlockSpec((1, 1, D), lambda i: (0, 0, 0)),
        ],
        out_specs=[
            pl.BlockSpec((MM_BLK, DH), lambda i: (i, 0)),
            pl.BlockSpec((MM_BLK, DH), lambda i: (i, 0)),
            pl.BlockSpec((1, 1, D), lambda i: (i, 0, 0)),
            pl.BlockSpec((1, 1, D), lambda i: (i, 0, 0)),
        ],
        out_shape=[
            jax.ShapeDtypeStruct((NP, DH), jnp.float32),
            jax.ShapeDtypeStruct((NP, DH), jnp.float32),
            jax.ShapeDtypeStruct((G, 1, D), jnp.float32),
            jax.ShapeDtypeStruct((G, 1, D), jnp.float32),
        ],
    )(x, W, al.reshape(1, 1, D), ar.reshape(1, 1, D))
    return hlo, hhi, el3.reshape(NP), er3.reshape(NP)


def _splat(v16, i):
    idx = jnp.broadcast_to(i, (16,)).astype(jnp.int32)[:, None]
    dnums = lax.GatherDimensionNumbers(
        offset_dims=(), collapsed_slice_dims=(0,), start_index_map=(0,))
    return lax.gather(v16, idx, dnums, (1,),
                      mode=lax.GatherScatterMode.PROMISE_IN_BOUNDS)


def _sc_body(do_relu,
             hlo_hbm, hhi_hbm, el_hbm, er_hbm, src_hbm, dst_hbm, b_hbm,
             out_hbm,
             el_t, er_t, src_t, dst_t, srcsel_t, ldst_t, w_t, rows_t,
             bias_t, den_t, obuf_t,
             acc_s, den_s):
    c = lax.axis_index("c")
    s = lax.axis_index("s")
    base = (c * HALF).astype(jnp.int32)
    zero16 = jnp.zeros((16,), jnp.float32)

    pltpu.sync_copy(el_hbm, el_t)
    pltpu.sync_copy(er_hbm, er_t)
    pltpu.sync_copy(b_hbm, bias_t)

    for dpass, h_hbm in enumerate((hlo_hbm, hhi_hbm)):
        for i in range(16):
            for j in range(DH // 16):
                obuf_t[i, pl.ds(j * 16, 16)] = zero16
        if dpass == 0:
            for g in range(ROWS_T // 16):
                den_t[pl.ds(g * 16, 16)] = zero16
            pltpu.sync_copy(den_t, den_s.at[pl.ds(s * ROWS_T, ROWS_T)])

        def _zrow(b, carry):
            pltpu.sync_copy(obuf_t, acc_s.at[pl.ds(s * ROWS_T + b * 16, 16)])
            return carry
        lax.fori_loop(0, ROWS_T // 16, _zrow, 0)

        plsc.subcore_barrier()

        for sub in range(NSUB):
            off = s * EC + sub * SUB
            pltpu.sync_copy(src_hbm.at[pl.ds(off, SUB)], src_t)
            pltpu.sync_copy(dst_hbm.at[pl.ds(off, SUB)], dst_t)

            def _kc_body(kc, carry):
                koff = kc * K
                for gg in range(K // 16):
                    go = koff + gg * 16
                    sv = src_t[pl.ds(go, 16)]
                    dv = dst_t[pl.ds(go, 16)]
                    elv = plsc.load_gather(el_t, [sv])
                    erv = plsc.load_gather(er_t, [dv])
                    e = elv + erv
                    e = jnp.where(e > 0, e, 0.2 * e)
                    ee = jnp.exp(e)
                    m = (dv >= base) & (dv < base + HALF)
                    w = jnp.where(m, ee, 0.0)
                    ld = jnp.where(m, dv - base, 0)
                    srcsel_t[pl.ds(gg * 16, 16)] = sv
                    ldst_t[pl.ds(gg * 16, 16)] = ld
                    w_t[pl.ds(gg * 16, 16)] = w
                if dpass == 0:
                    pltpu.sync_copy(w_t, den_s.at[ldst_t], add=True)
                pltpu.sync_copy(h_hbm.at[srcsel_t], rows_t)

                def _grow(g, carry2):
                    w16 = w_t[pl.ds(g * 16, 16)]

                    def _row(i, carry3):
                        ws = _splat(w16, i)
                        r = g * 16 + i
                        for j in range(DH // 16):
                            rows_t[r, pl.ds(j * 16, 16)] = (
                                rows_t[r, pl.ds(j * 16, 16)] * ws)
                        return carry3
                    return lax.fori_loop(0, 16, _row, carry2)
                lax.fori_loop(0, K // 16, _grow, 0)

                pltpu.sync_copy(rows_t, acc_s.at[ldst_t], add=True)
                return carry
            lax.fori_loop(0, NK, _kc_body, 0)

        plsc.subcore_barrier()

        if dpass == 0:
            pltpu.sync_copy(den_s.at[pl.ds(s * ROWS_T, ROWS_T)], den_t)
            for g in range(ROWS_T // 16):
                den_t[pl.ds(g * 16, 16)] = 1.0 / (den_t[pl.ds(g * 16, 16)]
                                                  + 1e-9)
        bv = [bias_t[pl.ds(dpass * DH + j * 16, 16)] for j in range(DH // 16)]
        gbase = c * HALF + s * ROWS_T

        def _ob(b, carry):
            pltpu.sync_copy(acc_s.at[pl.ds(s * ROWS_T + b * 16, 16)], obuf_t)
            iv16 = den_t[pl.ds(b * 16, 16)]

            def _row(i, carry2):
                ws = _splat(iv16, i)
                for j in range(DH // 16):
                    v = obuf_t[i, pl.ds(j * 16, 16)] * ws + bv[j]
                    if do_relu:
                        v = jnp.maximum(v, 0.0)
                    obuf_t[i, pl.ds(j * 16, 16)] = v
                return carry2
            lax.fori_loop(0, 16, _row, 0)
            pltpu.sync_copy(
                obuf_t,
                out_hbm.at[pl.ds(gbase + b * 16, 16), pl.ds(dpass * DH, DH)])
            return carry
        lax.fori_loop(0, ROWS_T // 16, _ob, 0)


def _sc_gat(hlo, hhi, el, er, src, dst, b, do_relu):
    mesh = plsc.VectorSubcoreMesh(core_axis_name="c", subcore_axis_name="s")
    f = pl.kernel(
        functools.partial(_sc_body, do_relu),
        out_type=jax.ShapeDtypeStruct((NP, D), jnp.float32),
        mesh=mesh,
        compiler_params=pltpu.CompilerParams(needs_layout_passes=False),
        scratch_types=[
            pltpu.VMEM((NP,), jnp.float32),
            pltpu.VMEM((NP,), jnp.float32),
            pltpu.VMEM((SUB,), jnp.int32),
            pltpu.VMEM((SUB,), jnp.int32),
            pltpu.VMEM((K,), jnp.int32),
            pltpu.VMEM((K,), jnp.int32),
            pltpu.VMEM((K,), jnp.float32),
            pltpu.VMEM((K, DH), jnp.float32),
            pltpu.VMEM((D,), jnp.float32),
            pltpu.VMEM((ROWS_T,), jnp.float32),
            pltpu.VMEM((16, DH), jnp.float32),
            pltpu.VMEM_SHARED((HALF, DH), jnp.float32),
            pltpu.VMEM_SHARED((HALF,), jnp.float32),
        ],
    )
    return f(hlo, hhi, el, er, src, dst, b)


def kernel(in_feat, edge_index, W1, al1, ar1, b1, W2, al2, ar2, b2,
           W3, al3, ar3, b3):
    src = edge_index[0]
    dst = edge_index[1]
    x = jnp.pad(in_feat, ((0, NP - N), (0, 0)))
    hlo, hhi, el, er = _tc_project(x, W1, al1, ar1)
    x = _sc_gat(hlo, hhi, el, er, src, dst, b1, True)
    hlo, hhi, el, er = _tc_project(x, W2, al2, ar2)
    x = _sc_gat(hlo, hhi, el, er, src, dst, b2, True)
    hlo, hhi, el, er = _tc_project(x, W3, al3, ar3)
    x = _sc_gat(hlo, hhi, el, er, src, dst, b3, False)
    return x[:N]
